# baseline (speedup 1.0000x reference)
import functools

import jax
import jax.numpy as jnp
from jax.experimental import pallas as pl
from jax.experimental.pallas import tpu as pltpu


def _ct2d_kernel(x4_ref, w_ref, b_ref, o_ref, y0_ref, y1_ref):
    C = o_ref.shape[1]
    OW = o_ref.shape[3]
    bh = o_ref.shape[2] // 2
    bias = b_ref[...].reshape(C, 1, 1)

    n_sl = bh * OW
    s0 = x4_ref[0, :, pl.ds(0, n_sl)]
    s1 = x4_ref[0, :, pl.ds(OW, n_sl)]
    s2 = x4_ref[0, :, pl.ds(2 * OW, n_sl)]
    y0_ref[...] = jnp.dot(w_ref[0], s0, preferred_element_type=jnp.float32)
    y0_ref[...] += jnp.dot(w_ref[1], s1, preferred_element_type=jnp.float32)
    y1_ref[...] = jnp.dot(w_ref[2], s1, preferred_element_type=jnp.float32)
    y1_ref[...] += jnp.dot(w_ref[3], s2, preferred_element_type=jnp.float32)

    def write_grp(g, carry):
        v0 = y0_ref[:, pl.ds(g * 4 * OW, 4 * OW)].reshape(C, 4, OW) + bias
        v1 = y1_ref[:, pl.ds(g * 4 * OW, 4 * OW)].reshape(C, 4, OW) + bias
        v = jnp.stack([v0, v1], axis=2).reshape(C, 8, OW)
        o_ref[0, :, pl.ds(8 * g, 8), :] = v
        return carry

    jax.lax.fori_loop(0, bh // 4, write_grp, 0, unroll=2)


@jax.jit
def _forward(x_nchw, weight, bias):
    N, C, H, W = x_nchw.shape
    OH, OW = 2 * H, 2 * W

    z = jnp.zeros_like(x_nchw)
    x_l = jnp.pad(x_nchw[:, :, :, 1:], ((0, 0), (0, 0), (0, 0), (0, 1)))
    x_r = jnp.pad(x_nchw[:, :, :, :-1], ((0, 0), (0, 0), (0, 0), (1, 0)))
    taps = [
        jnp.stack([x_r, z], axis=-1),
        jnp.stack([z, x_nchw], axis=-1),
        jnp.stack([x_nchw, z], axis=-1),
        jnp.stack([z, x_l], axis=-1),
    ]
    x4 = jnp.stack([t.reshape(N, C, H, OW) for t in taps], axis=1)
    x4 = jnp.pad(x4, ((0, 0), (0, 0), (0, 0), (1, 1), (0, 0)))
    x4 = x4.reshape(N, 4 * C, (H + 2) * OW).astype(jnp.bfloat16)

    wp = []
    for dy in (0, 1):
        for di in (0, 1):
            kh = 3 - dy - 2 * di
            tp = [weight[:, :, kh, 3 - kw] for kw in range(4)]
            wp.append(jnp.stack(tp, axis=0).reshape(4 * C, C).T)
    w_all = jnp.stack(wp, axis=0).astype(jnp.bfloat16)
    b2d = bias.reshape(C, 1).astype(jnp.float32)

    return pl.pallas_call(
        _ct2d_kernel,
        out_shape=jax.ShapeDtypeStruct((N, C, OH, OW), x_nchw.dtype),
        grid=(N,),
        in_specs=[
            pl.BlockSpec((1, 4 * C, (H + 2) * OW), lambda n: (n, 0, 0)),
            pl.BlockSpec((4, C, 4 * C), lambda n: (0, 0, 0)),
            pl.BlockSpec((C, 1), lambda n: (0, 0)),
        ],
        out_specs=pl.BlockSpec((1, C, OH, OW), lambda n: (n, 0, 0, 0)),
        scratch_shapes=[
            pltpu.VMEM((C, H * OW), jnp.float32),
            pltpu.VMEM((C, H * OW), jnp.float32),
        ],
        compiler_params=pltpu.CompilerParams(
            dimension_semantics=("parallel",)),
    )(x4, w_all, b2d)


def kernel(x_nchw, weight, bias):
    return _forward(x_nchw, weight, bias)


# speedup vs baseline: 2.7398x; 2.7398x over previous
import functools

import jax
import jax.numpy as jnp
from jax import lax
from jax.experimental import pallas as pl
from jax.experimental.pallas import tpu as pltpu


def _ct2d_kernel(xw_ref, w_ref, b_ref, o_ref, a_ref, y0_ref, y1_ref):
    C = xw_ref.shape[2]
    OW = o_ref.shape[3]
    bh = o_ref.shape[2] // 2
    a0 = pl.program_id(1) * bh
    bias = b_ref[...].reshape(C, 1, 1)

    def build_row(t, carry):
        row = xw_ref[0, a0 + t, :, :]
        for kw in range(4):
            a_ref[pl.ds(kw * C, C), pl.ds(t * OW, OW)] = row[:, kw:kw + OW]
        return carry

    lax.fori_loop(0, bh + 2, build_row, 0, unroll=2)

    n_sl = bh * OW
    s0 = a_ref[:, pl.ds(0, n_sl)]
    s1 = a_ref[:, pl.ds(OW, n_sl)]
    s2 = a_ref[:, pl.ds(2 * OW, n_sl)]
    y0_ref[...] = jnp.dot(w_ref[0], s0, preferred_element_type=jnp.float32)
    y0_ref[...] += jnp.dot(w_ref[1], s1, preferred_element_type=jnp.float32)
    y1_ref[...] = jnp.dot(w_ref[2], s1, preferred_element_type=jnp.float32)
    y1_ref[...] += jnp.dot(w_ref[3], s2, preferred_element_type=jnp.float32)

    def write_grp(g, carry):
        u = jnp.concatenate(
            [r[:, pl.ds((g * 4 + j) * OW, OW)]
             for j in range(4) for r in (y0_ref, y1_ref)], axis=1)
        o_ref[0, :, pl.ds(8 * g, 8), :] = u.reshape(C, 8, OW) + bias
        return carry

    lax.fori_loop(0, bh // 4, write_grp, 0, unroll=2)


@functools.partial(jax.jit, static_argnames=("block_h",))
def _forward(x_nchw, weight, bias, *, block_h=64):
    N, C, H, W = x_nchw.shape
    OH, OW = 2 * H, 2 * W
    WD = 2 * W + 3

    bh = block_h
    while H % bh:
        bh //= 2
    n_hb = H // bh

    xt = jnp.transpose(x_nchw, (0, 2, 1, 3))
    x_il = jnp.stack([xt, jnp.zeros_like(xt)], axis=-1).reshape(N, H, C, 2 * W)
    xw = jnp.pad(x_il, ((0, 0), (1, 1), (0, 0), (2, 1))).astype(jnp.bfloat16)

    wp = []
    for dy in (0, 1):
        for di in (0, 1):
            kh = 3 - dy - 2 * di
            taps = [weight[:, :, kh, 3 - kw] for kw in range(4)]
            wp.append(jnp.stack(taps, axis=0).reshape(4 * C, C).T)
    w_all = jnp.stack(wp, axis=0).astype(jnp.bfloat16)
    b2d = bias.reshape(C, 1).astype(jnp.float32)

    return pl.pallas_call(
        _ct2d_kernel,
        out_shape=jax.ShapeDtypeStruct((N, C, OH, OW), x_nchw.dtype),
        grid=(N, n_hb),
        in_specs=[
            pl.BlockSpec((1, H + 2, C, WD), lambda n, h: (n, 0, 0, 0)),
            pl.BlockSpec((4, C, 4 * C), lambda n, h: (0, 0, 0)),
            pl.BlockSpec((C, 1), lambda n, h: (0, 0)),
        ],
        out_specs=pl.BlockSpec((1, C, 2 * bh, OW), lambda n, h: (n, 0, h, 0)),
        scratch_shapes=[
            pltpu.VMEM((4 * C, (bh + 2) * OW), jnp.bfloat16),
            pltpu.VMEM((C, bh * OW), jnp.float32),
            pltpu.VMEM((C, bh * OW), jnp.float32),
        ],
        compiler_params=pltpu.CompilerParams(
            dimension_semantics=("parallel", "parallel")),
    )(xw, w_all, b2d)


def kernel(x_nchw, weight, bias):
    return _forward(x_nchw, weight, bias)


# speedup vs baseline: 2.7785x; 1.0141x over previous
import functools

import jax
import jax.numpy as jnp
from jax import lax
from jax.experimental import pallas as pl
from jax.experimental.pallas import tpu as pltpu


def _ct2d_kernel(xw_ref, w_ref, b_ref, o_ref, a_ref, y0_ref, y1_ref):
    C = xw_ref.shape[2]
    OW = o_ref.shape[3]
    bh = o_ref.shape[2] // 2
    a0 = pl.program_id(1) * bh
    bias = b_ref[...].reshape(C, 1, 1)

    def build_row(t, carry):
        row = xw_ref[0, a0 + t, :, :]
        a_ref[:, pl.ds(t * OW, OW)] = jnp.concatenate(
            [row[:, kw:kw + OW] for kw in range(4)], axis=0)
        return carry

    lax.fori_loop(0, bh + 2, build_row, 0, unroll=2)

    n_sl = bh * OW
    s0 = a_ref[:, pl.ds(0, n_sl)]
    s1 = a_ref[:, pl.ds(OW, n_sl)]
    s2 = a_ref[:, pl.ds(2 * OW, n_sl)]
    y0_ref[...] = jnp.dot(w_ref[0], s0, preferred_element_type=jnp.float32)
    y0_ref[...] += jnp.dot(w_ref[1], s1, preferred_element_type=jnp.float32)
    y1_ref[...] = jnp.dot(w_ref[2], s1, preferred_element_type=jnp.float32)
    y1_ref[...] += jnp.dot(w_ref[3], s2, preferred_element_type=jnp.float32)

    def write_grp(g, carry):
        u = jnp.concatenate(
            [r[:, pl.ds((g * 4 + j) * OW, OW)]
             for j in range(4) for r in (y0_ref, y1_ref)], axis=1)
        o_ref[0, :, pl.ds(8 * g, 8), :] = u.reshape(C, 8, OW) + bias
        return carry

    lax.fori_loop(0, bh // 4, write_grp, 0, unroll=2)


@functools.partial(jax.jit, static_argnames=("block_h",))
def _forward(x_nchw, weight, bias, *, block_h=64):
    N, C, H, W = x_nchw.shape
    OH, OW = 2 * H, 2 * W
    WD = 2 * W + 3

    bh = block_h
    while H % bh:
        bh //= 2
    n_hb = H // bh

    xt = jnp.transpose(x_nchw, (0, 2, 1, 3))
    x_il = jnp.stack([xt, jnp.zeros_like(xt)], axis=-1).reshape(N, H, C, 2 * W)
    xw = jnp.pad(x_il, ((0, 0), (1, 1), (0, 0), (2, 1))).astype(jnp.bfloat16)

    wp = []
    for dy in (0, 1):
        for di in (0, 1):
            kh = 3 - dy - 2 * di
            taps = [weight[:, :, kh, 3 - kw] for kw in range(4)]
            wp.append(jnp.stack(taps, axis=0).reshape(4 * C, C).T)
    w_all = jnp.stack(wp, axis=0).astype(jnp.bfloat16)
    b2d = bias.reshape(C, 1).astype(jnp.float32)

    return pl.pallas_call(
        _ct2d_kernel,
        out_shape=jax.ShapeDtypeStruct((N, C, OH, OW), x_nchw.dtype),
        grid=(N, n_hb),
        in_specs=[
            pl.BlockSpec((1, H + 2, C, WD), lambda n, h: (n, 0, 0, 0)),
            pl.BlockSpec((4, C, 4 * C), lambda n, h: (0, 0, 0)),
            pl.BlockSpec((C, 1), lambda n, h: (0, 0)),
        ],
        out_specs=pl.BlockSpec((1, C, 2 * bh, OW), lambda n, h: (n, 0, h, 0)),
        scratch_shapes=[
            pltpu.VMEM((4 * C, (bh + 2) * OW), jnp.bfloat16),
            pltpu.VMEM((C, bh * OW), jnp.float32),
            pltpu.VMEM((C, bh * OW), jnp.float32),
        ],
        compiler_params=pltpu.CompilerParams(
            dimension_semantics=("parallel", "parallel")),
    )(xw, w_all, b2d)


def kernel(x_nchw, weight, bias):
    return _forward(x_nchw, weight, bias)
